# baseline (speedup 1.0000x reference)
import jax
import jax.numpy as jnp
from jax.experimental import pallas as pl


def _enc_block(x_ref, w_ref, b_ref, o_ref):
    xs = x_ref[...]
    m = jnp.sum(xs, axis=1) * (1.0 / xs.shape[1])
    z = jnp.dot(m, w_ref[...], preferred_element_type=jnp.float32)
    z = z + b_ref[...]
    o_ref[...] = jnp.broadcast_to(z[:, None, :], o_ref.shape)


def kernel(x, W, b):
    B, T, K, S = x.shape
    Z = W.shape[1]
    N = B * T
    xf = x.reshape(N, K, S)
    G = 128
    grid = (N // G,)
    out = pl.pallas_call(
        _enc_block,
        grid=grid,
        in_specs=[
            pl.BlockSpec((G, K, S), lambda i: (i, 0, 0)),
            pl.BlockSpec((S, Z), lambda i: (0, 0)),
            pl.BlockSpec((1, Z), lambda i: (0, 0)),
        ],
        out_specs=pl.BlockSpec((G, K, Z), lambda i: (i, 0, 0)),
        out_shape=jax.ShapeDtypeStruct((N, K, Z), jnp.float32),
    )(xf, W, b.reshape(1, Z))
    return out.reshape(B, T, K, Z)
